# baseline (speedup 1.0000x reference)
import functools
import jax
import jax.numpy as jnp
from jax import lax
from jax.experimental import pallas as pl
from jax.experimental.pallas import tpu as pltpu
from jax.experimental.pallas import tpu_sc as plsc

BATCH, SEQ, MODEL_DIM = 4, 4096, 1024
ROWS = BATCH * SEQ
NC, NS, L = 2, 16, 16
NW = NC * NS
RPW = ROWS // NW
NVEC = RPW // L
CHUNK = L
NBUF = 6

_mesh = plsc.VectorSubcoreMesh(core_axis_name="c", subcore_axis_name="s")


@functools.partial(
    pl.kernel, mesh=_mesh,
    compiler_params=pltpu.CompilerParams(needs_layout_passes=False),
    out_type=jax.ShapeDtypeStruct((ROWS, MODEL_DIM), jnp.float32),
    scratch_types=[
        pltpu.VMEM((RPW,), jnp.int32),
        pltpu.VMEM((RPW + L,), jnp.int32),
        pltpu.VMEM((RPW + L,), jnp.int32),
        pltpu.VMEM((NBUF, CHUNK, MODEL_DIM), jnp.float32),
        pltpu.VMEM((CHUNK, MODEL_DIM), jnp.float32),
        [pltpu.SemaphoreType.DMA] * NBUF,
        [pltpu.SemaphoreType.DMA] * NBUF,
        pltpu.SemaphoreType.DMA,
    ],
)
def _sc_masker(seqs_hbm, mask_hbm, embed_hbm, out_hbm,
               maskv, idxu, idxm, ring, embedbuf,
               sems_g, sems_s, sem_m):
    wid = lax.axis_index("s") * NC + lax.axis_index("c")
    base = wid * RPW

    pltpu.sync_copy(mask_hbm.at[pl.ds(base, RPW)], maskv)
    zidx = jnp.zeros((L,), jnp.int32)
    pltpu.async_copy(embed_hbm.at[zidx], embedbuf, sem_m)

    def comp_body(i, carry):
        cu, cm = carry
        mv = maskv[pl.ds(i * L, L)]
        mm = mv != 0
        mu = jnp.logical_not(mm)
        vals = base + i * L + lax.iota(jnp.int32, L)
        plsc.store_compressed(idxu.at[pl.ds(cu, L)], vals, mask=mu)
        plsc.store_compressed(idxm.at[pl.ds(cm, L)], vals, mask=mm)
        nu = jnp.max(plsc.all_reduce_population_count(mu))
        return (cu + nu, cm + (L - nu))

    cu, cm = lax.fori_loop(0, NVEC, comp_body,
                           (jnp.int32(0), jnp.int32(0)))

    u0 = idxu[pl.ds(0, L)][0]
    m0 = idxm[pl.ds(0, L)][0]
    idxu[pl.ds(cu, L)] = jnp.full((L,), u0, jnp.int32)
    idxm[pl.ds(cm, L)] = jnp.full((L,), m0, jnp.int32)

    ncu = (cu + CHUNK - 1) // CHUNK
    ncm = (cm + CHUNK - 1) // CHUNK

    pltpu.make_async_copy(seqs_hbm.at[pl.ds(0, CHUNK)],
                          embedbuf, sem_m).wait()

    def fire_body(j, _):
        iv = idxm[pl.ds(j * CHUNK, CHUNK)]
        pltpu.async_copy(embedbuf, out_hbm.at[iv], sem_m)
        return 0

    lax.fori_loop(0, ncm, fire_body, 0)

    def gather(j, slot):
        iv = idxu[pl.ds(j * CHUNK, CHUNK)]
        pltpu.async_copy(seqs_hbm.at[iv], ring.at[slot], sems_g[slot])

    def wait_gather(slot):
        pltpu.make_async_copy(seqs_hbm.at[pl.ds(0, CHUNK)],
                              ring.at[slot], sems_g[slot]).wait()

    def scatter(j, slot):
        iv = idxu[pl.ds(j * CHUNK, CHUNK)]
        pltpu.async_copy(ring.at[slot], out_hbm.at[iv], sems_s[slot])

    def wait_scatter(slot):
        pltpu.make_async_copy(ring.at[slot],
                              out_hbm.at[pl.ds(0, CHUNK)], sems_s[slot]).wait()

    for p in range(NBUF - 1):
        @pl.when(p < ncu)
        def _(p=p):
            gather(p, p)

    def copy_group(g, _):
        for b in range(NBUF):
            j = g * NBUF + b

            @pl.when(j < ncu)
            def _(b=b, j=j):
                wait_gather(b)
                scatter(j, b)
                nslot = (b + NBUF - 1) % NBUF

                @pl.when(j + NBUF - 1 < ncu)
                def _():
                    @pl.when(j + NBUF - 1 >= NBUF)
                    def _():
                        wait_scatter(nslot)
                    gather(j + NBUF - 1, nslot)
        return 0

    lax.fori_loop(0, (ncu + NBUF - 1) // NBUF, copy_group, 0)

    for s in range(NBUF):
        @pl.when(s < ncu)
        def _(s=s):
            wait_scatter(s)

    def drain_m(j, _):
        pltpu.make_async_copy(embedbuf, out_hbm.at[pl.ds(0, CHUNK)],
                              sem_m).wait()
        return 0

    lax.fori_loop(0, ncm, drain_m, 0)


def kernel(seqs, temporal_mask, temporal_mask_embed):
    seqs2 = seqs.reshape(ROWS, MODEL_DIM)
    mask_i = temporal_mask.reshape(ROWS).astype(jnp.int32)
    embed2 = temporal_mask_embed.reshape(1, MODEL_DIM)
    out = _sc_masker(seqs2, mask_i, embed2)
    return (out.reshape(BATCH, SEQ, MODEL_DIM), temporal_mask)


# speedup vs baseline: 1.9499x; 1.9499x over previous
import jax
import jax.numpy as jnp
from jax.experimental import pallas as pl
from jax.experimental.pallas import tpu as pltpu

BATCH, SEQ, MODEL_DIM = 4, 4096, 1024
ROWS = BATCH * SEQ
G0 = ROWS // 128
MROWS = 8


def _body(m_ref, s_ref, e_ref, o_ref):
    mt = m_ref[...].T
    e = e_ref[...]
    for j in range(MROWS):
        mj = mt[:, j:j + 1]
        o_ref[j] = jnp.where(mj, e, s_ref[j])


def kernel(seqs, temporal_mask, temporal_mask_embed):
    mask2d = temporal_mask.reshape(G0, 128)
    seqs3 = seqs.reshape(G0, 128, MODEL_DIM)
    embed2d = temporal_mask_embed.reshape(1, MODEL_DIM)

    out = pl.pallas_call(
        _body,
        grid=(G0 // MROWS,),
        in_specs=[
            pl.BlockSpec((MROWS, 128), lambda i: (i, 0)),
            pl.BlockSpec((MROWS, 128, MODEL_DIM), lambda i: (i, 0, 0)),
            pl.BlockSpec((1, MODEL_DIM), lambda i: (0, 0)),
        ],
        out_specs=pl.BlockSpec((MROWS, 128, MODEL_DIM), lambda i: (i, 0, 0)),
        out_shape=jax.ShapeDtypeStruct((G0, 128, MODEL_DIM), seqs.dtype),
    )(mask2d, seqs3, embed2d)
    return (out.reshape(BATCH, SEQ, MODEL_DIM), temporal_mask)


# speedup vs baseline: 2.0087x; 1.0301x over previous
import jax
import jax.numpy as jnp
from jax.experimental import pallas as pl
from jax.experimental.pallas import tpu as pltpu

BATCH, SEQ, MODEL_DIM = 4, 4096, 1024
ROWS = BATCH * SEQ
G0 = ROWS // 128
MROWS = 16


def _body(m_ref, s_ref, e_ref, o_ref):
    mt = m_ref[...].T
    e = e_ref[...]
    for j in range(MROWS):
        mj = mt[:, j:j + 1]
        o_ref[j] = jnp.where(mj, e, s_ref[j])


def kernel(seqs, temporal_mask, temporal_mask_embed):
    mask2d = temporal_mask.reshape(G0, 128)
    seqs3 = seqs.reshape(G0, 128, MODEL_DIM)
    embed2d = temporal_mask_embed.reshape(1, MODEL_DIM)

    out = pl.pallas_call(
        _body,
        grid=(G0 // MROWS,),
        in_specs=[
            pl.BlockSpec((MROWS, 128), lambda i: (i, 0)),
            pl.BlockSpec((MROWS, 128, MODEL_DIM), lambda i: (i, 0, 0)),
            pl.BlockSpec((1, MODEL_DIM), lambda i: (0, 0)),
        ],
        out_specs=pl.BlockSpec((MROWS, 128, MODEL_DIM), lambda i: (i, 0, 0)),
        out_shape=jax.ShapeDtypeStruct((G0, 128, MODEL_DIM), seqs.dtype),
    )(mask2d, seqs3, embed2d)
    return (out.reshape(BATCH, SEQ, MODEL_DIM), temporal_mask)
